# Initial kernel scaffold; baseline (speedup 1.0000x reference)
import jax
import jax.numpy as jnp
from jax.experimental import pallas as pl


def kernel(scores):
    raise NotImplementedError("write your pallas kernel here")


# speedup vs baseline: 1.3861x; 1.3861x over previous
import jax
import jax.numpy as jnp
import numpy as np
from jax.experimental import pallas as pl

_EPSILON = float(np.finfo(np.float32).tiny)
_K = 32
_TAU = 0.1
_ROWS_PER_BLOCK = 8


def _gumbel_topk_block(s_ref, g_ref, out_ref):
    s0 = s_ref[...] + g_ref[...]
    zeros = jnp.zeros_like(s0)

    def soft_iter(_, carry):
        s, khot, onehot = carry
        khot_mask = jnp.maximum(1.0 - onehot, _EPSILON)
        s = s + jnp.log(khot_mask)
        onehot = jax.nn.softmax(s / _TAU, axis=1)
        return s, khot + onehot, onehot

    _, khot, _ = jax.lax.fori_loop(
        0, _K, soft_iter, (s0, zeros, zeros))

    iota = jax.lax.broadcasted_iota(jnp.int32, khot.shape, 1)

    def sel_iter(_, carry):
        work, hard = carry
        m = jnp.max(work, axis=1, keepdims=True)
        idx = jnp.min(
            jnp.where(work == m, iota, jnp.int32(np.iinfo(np.int32).max)),
            axis=1, keepdims=True)
        pick = iota == idx
        hard = jnp.where(pick, 1.0, hard)
        work = jnp.where(pick, -jnp.inf, work)
        return work, hard

    _, hard = jax.lax.fori_loop(0, _K, sel_iter, (khot, zeros))
    out_ref[...] = (hard - khot) + khot


def kernel(scores):
    bsz, nmax, _, ens = scores.shape
    n = nmax * nmax
    s2 = jnp.transpose(scores, (0, 3, 1, 2)).reshape(bsz * ens, n)
    g = jax.random.gumbel(jax.random.key(42), s2.shape, dtype=s2.dtype)
    r = _ROWS_PER_BLOCK
    out = pl.pallas_call(
        _gumbel_topk_block,
        grid=(s2.shape[0] // r,),
        in_specs=[
            pl.BlockSpec((r, n), lambda i: (i, 0)),
            pl.BlockSpec((r, n), lambda i: (i, 0)),
        ],
        out_specs=pl.BlockSpec((r, n), lambda i: (i, 0)),
        out_shape=jax.ShapeDtypeStruct(s2.shape, s2.dtype),
    )(s2, g)
    res = out.reshape(bsz, ens, nmax, nmax)
    return jnp.transpose(res, (0, 2, 3, 1))


# speedup vs baseline: 2.0168x; 1.4550x over previous
import jax
import jax.numpy as jnp
import numpy as np
from jax.experimental import pallas as pl

_EPSILON = float(np.finfo(np.float32).tiny)
_K = 32
_TAU = 0.1
_ROWS_PER_BLOCK = 8


def _gumbel_topk_block(s_ref, g_ref, out_ref):
    s0 = s_ref[...] + g_ref[...]
    zeros = jnp.zeros_like(s0)

    def soft_iter(_, carry):
        s, khot, onehot = carry
        khot_mask = jnp.maximum(1.0 - onehot, _EPSILON)
        s = s + jnp.log(khot_mask)
        onehot = jax.nn.softmax(s / _TAU, axis=1)
        return s, khot + onehot, onehot

    _, khot, _ = jax.lax.fori_loop(
        0, _K, soft_iter, (s0, zeros, zeros))

    ki = jax.lax.bitcast_convert_type(khot, jnp.int32)
    rows = ki.shape[0]

    def bit_iter(b, t):
        cand = t | (jnp.int32(1) << (30 - b))
        cnt = jnp.sum((ki >= cand).astype(jnp.int32), axis=1, keepdims=True)
        return jnp.where(cnt >= _K, cand, t)

    t = jax.lax.fori_loop(0, 31, bit_iter, jnp.zeros((rows, 1), jnp.int32))

    ge = ki >= t
    val = (1.0 - khot) + khot
    cnt_ge = jnp.sum(ge.astype(jnp.int32), axis=1, keepdims=True)

    def no_ties():
        return jnp.where(ge, val, 0.0)

    def with_ties():
        iota = jax.lax.broadcasted_iota(jnp.int32, ki.shape, 1)
        gt = ki > t
        eq = jnp.logical_and(ge, jnp.logical_not(gt))
        need = _K - jnp.sum(gt.astype(jnp.int32), axis=1, keepdims=True)

        def idx_iter(b, p):
            cand = p + (jnp.int32(1) << (13 - b))
            f = jnp.sum(jnp.logical_and(eq, iota <= cand).astype(jnp.int32),
                        axis=1, keepdims=True)
            return jnp.where(f <= need - 1, cand, p)

        p = jax.lax.fori_loop(0, 14, idx_iter,
                              jnp.full((rows, 1), -1, jnp.int32))
        sel = jnp.logical_or(gt, jnp.logical_and(eq, iota <= p + 1))
        return jnp.where(sel, val, 0.0)

    out_ref[...] = jax.lax.cond(jnp.all(cnt_ge == _K), no_ties, with_ties)


def kernel(scores):
    bsz, nmax, _, ens = scores.shape
    n = nmax * nmax
    s2 = jnp.transpose(scores, (0, 3, 1, 2)).reshape(bsz * ens, n)
    g = jax.random.gumbel(jax.random.key(42), s2.shape, dtype=s2.dtype)
    r = _ROWS_PER_BLOCK
    out = pl.pallas_call(
        _gumbel_topk_block,
        grid=(s2.shape[0] // r,),
        in_specs=[
            pl.BlockSpec((r, n), lambda i: (i, 0)),
            pl.BlockSpec((r, n), lambda i: (i, 0)),
        ],
        out_specs=pl.BlockSpec((r, n), lambda i: (i, 0)),
        out_shape=jax.ShapeDtypeStruct(s2.shape, s2.dtype),
    )(s2, g)
    res = out.reshape(bsz, ens, nmax, nmax)
    return jnp.transpose(res, (0, 2, 3, 1))
